# baseline (speedup 1.0000x reference)
import functools

import jax
import jax.numpy as jnp
from jax import lax
from jax.experimental import pallas as pl
from jax.experimental.pallas import tpu as pltpu
from jax.experimental.pallas import tpu_sc as plsc

N_NODES = 10000
N_EDGES = 320000
D_FEAT = 128

B = N_EDGES * 2
NW = 32
PER_W = B // NW
CHUNK = 80
N_CHUNKS = PER_W // CHUNK
NBUF = 2
N_PARTS = 5
PART = N_CHUNKS // N_PARTS
N_GROUPS = PART // NBUF
STAGE_ROWS = 1000


def _sc_gather(table, idx4d):
    mesh = plsc.VectorSubcoreMesh(core_axis_name="c", subcore_axis_name="s")

    @functools.partial(
        pl.kernel,
        mesh=mesh,
        out_type=jax.ShapeDtypeStruct((B, D_FEAT), jnp.float32),
        scratch_types=[
            pltpu.VMEM_SHARED((N_NODES, D_FEAT), jnp.float32),
            pltpu.VMEM((PART, CHUNK), jnp.int32),
            pltpu.VMEM((NBUF, CHUNK, D_FEAT), jnp.float32),
            pltpu.SemaphoreType.DMA,
            pltpu.SemaphoreType.DMA,
            pltpu.SemaphoreType.DMA,
        ],
    )
    def k(table_hbm, idx_hbm, out_hbm, table_s, idx_v, rows_v, gsem_s, gsem_h, wsem):
        cid = lax.axis_index("c")
        sid = lax.axis_index("s")
        wid = sid * 2 + cid

        @pl.when(sid < N_NODES // STAGE_ROWS)
        def _():
            pltpu.sync_copy(
                table_hbm.at[pl.ds(sid * STAGE_ROWS, STAGE_ROWS)],
                table_s.at[pl.ds(sid * STAGE_ROWS, STAGE_ROWS)],
            )

        pltpu.sync_copy(idx_hbm.at[wid, 0], idx_v)
        plsc.subcore_barrier()

        def src(b, j):
            if b % 2 == 0:
                return table_s.at[idx_v.at[j]], gsem_s
            return table_hbm.at[idx_v.at[j]], gsem_h

        def run_part(p):
            out0 = wid * PER_W + p * PART * CHUNK

            for b in range(NBUF):
                ref, sem = src(b, b)
                pltpu.async_copy(ref, rows_v.at[b], sem)

            def body(g, carry):
                base = g * NBUF
                for b in range(NBUF):
                    j = base + b
                    ref, sem = src(b, j)
                    pltpu.make_async_copy(ref, rows_v.at[b], sem).wait()
                    pltpu.async_copy(
                        rows_v.at[b],
                        out_hbm.at[pl.ds(out0 + j * CHUNK, CHUNK)],
                        wsem,
                    )
                for b in range(NBUF):
                    j = base + b
                    pltpu.make_async_copy(
                        rows_v.at[b],
                        out_hbm.at[pl.ds(out0 + j * CHUNK, CHUNK)],
                        wsem,
                    ).wait()

                    @pl.when(g < N_GROUPS - 1)
                    def _():
                        ref, sem = src(b, j + NBUF)
                        pltpu.async_copy(ref, rows_v.at[b], sem)

                return carry

            lax.fori_loop(0, N_GROUPS, body, 0)

        run_part(0)
        for p in range(1, N_PARTS):
            pltpu.sync_copy(idx_hbm.at[wid, p], idx_v)
            run_part(p)

    return k(table, idx4d)


def kernel(node_features, edge_list):
    idx4d = edge_list.astype(jnp.int32).reshape(NW, N_PARTS, PART, CHUNK)
    out = _sc_gather(node_features, idx4d)
    return out.reshape(N_EDGES, 2, D_FEAT)


# speedup vs baseline: 1.3316x; 1.3316x over previous
import functools

import jax
import jax.numpy as jnp
from jax import lax
from jax.experimental import pallas as pl
from jax.experimental.pallas import tpu as pltpu
from jax.experimental.pallas import tpu_sc as plsc

N_NODES = 10000
N_EDGES = 320000
D_FEAT = 128

B = N_EDGES * 2
NW = 32
PER_W = B // NW
CHUNK = 80
N_CHUNKS = PER_W // CHUNK
NBUF = 2
N_PARTS = 5
PART = N_CHUNKS // N_PARTS
N_GROUPS = PART // NBUF
STAGE_ROWS = 1000


def _sc_gather(table, idx4d):
    mesh = plsc.VectorSubcoreMesh(core_axis_name="c", subcore_axis_name="s")

    @functools.partial(
        pl.kernel,
        mesh=mesh,
        out_type=jax.ShapeDtypeStruct((B, D_FEAT), jnp.float32),
        scratch_types=[
            pltpu.VMEM_SHARED((N_NODES, D_FEAT), jnp.float32),
            pltpu.VMEM((PART, CHUNK), jnp.int32),
            pltpu.VMEM((NBUF, CHUNK, D_FEAT), jnp.float32),
            pltpu.SemaphoreType.DMA,
            pltpu.SemaphoreType.DMA,
        ],
    )
    def k(table_hbm, idx_hbm, out_hbm, table_s, idx_v, rows_v, gsem, wsem):
        cid = lax.axis_index("c")
        sid = lax.axis_index("s")
        wid = sid * 2 + cid

        @pl.when(sid < N_NODES // STAGE_ROWS)
        def _():
            pltpu.sync_copy(
                table_hbm.at[pl.ds(sid * STAGE_ROWS, STAGE_ROWS)],
                table_s.at[pl.ds(sid * STAGE_ROWS, STAGE_ROWS)],
            )

        pltpu.sync_copy(idx_hbm.at[wid, 0], idx_v)
        plsc.subcore_barrier()

        def src(b, j):
            return table_s.at[idx_v.at[j]], gsem

        def run_part(p):
            out0 = wid * PER_W + p * PART * CHUNK

            for b in range(NBUF):
                ref, sem = src(b, b)
                pltpu.async_copy(ref, rows_v.at[b], sem)

            def body(g, carry):
                base = g * NBUF
                for b in range(NBUF):
                    j = base + b
                    ref, sem = src(b, j)
                    pltpu.make_async_copy(ref, rows_v.at[b], sem).wait()
                    pltpu.async_copy(
                        rows_v.at[b],
                        out_hbm.at[pl.ds(out0 + j * CHUNK, CHUNK)],
                        wsem,
                    )
                for b in range(NBUF):
                    j = base + b
                    pltpu.make_async_copy(
                        rows_v.at[b],
                        out_hbm.at[pl.ds(out0 + j * CHUNK, CHUNK)],
                        wsem,
                    ).wait()

                    @pl.when(g < N_GROUPS - 1)
                    def _():
                        ref, sem = src(b, j + NBUF)
                        pltpu.async_copy(ref, rows_v.at[b], sem)

                return carry

            lax.fori_loop(0, N_GROUPS, body, 0)

        run_part(0)
        for p in range(1, N_PARTS):
            pltpu.sync_copy(idx_hbm.at[wid, p], idx_v)
            run_part(p)

    return k(table, idx4d)


def kernel(node_features, edge_list):
    idx4d = edge_list.astype(jnp.int32).reshape(NW, N_PARTS, PART, CHUNK)
    out = _sc_gather(node_features, idx4d)
    return out.reshape(N_EDGES, 2, D_FEAT)
